# baseline (speedup 1.0000x reference)
import jax
import jax.numpy as jnp
from jax.experimental import pallas as pl
from jax.experimental.pallas import tpu as pltpu


def _copy_body(x_ref, r_ref, d_ref, xo_ref, ro_ref, do_ref,
               sem_x, sem_r, sem_d):
    cx = pltpu.make_async_copy(x_ref, xo_ref, sem_x)
    cr = pltpu.make_async_copy(r_ref, ro_ref, sem_r)
    cd = pltpu.make_async_copy(d_ref, do_ref, sem_d)
    cx.start()
    cr.start()
    cd.start()
    cx.wait()
    cr.wait()
    cd.wait()


def kernel(sampled_point_xyz, sampled_point_ray_direction, sampled_point_distance):
    n = sampled_point_xyz.shape[0]
    pos, ray, dists = pl.pallas_call(
        _copy_body,
        in_specs=[
            pl.BlockSpec(memory_space=pl.ANY),
            pl.BlockSpec(memory_space=pl.ANY),
            pl.BlockSpec(memory_space=pl.ANY),
        ],
        out_specs=[
            pl.BlockSpec(memory_space=pl.ANY),
            pl.BlockSpec(memory_space=pl.ANY),
            pl.BlockSpec(memory_space=pl.ANY),
        ],
        out_shape=[
            jax.ShapeDtypeStruct((n, 3), jnp.float32),
            jax.ShapeDtypeStruct((n, 3), jnp.float32),
            jax.ShapeDtypeStruct((n,), jnp.float32),
        ],
        scratch_shapes=[
            pltpu.SemaphoreType.DMA,
            pltpu.SemaphoreType.DMA,
            pltpu.SemaphoreType.DMA,
        ],
    )(sampled_point_xyz, sampled_point_ray_direction, sampled_point_distance)
    return (pos, ray, dists)


# speedup vs baseline: 7.3705x; 7.3705x over previous
import jax
import jax.numpy as jnp
from jax.experimental import pallas as pl
from jax.experimental.pallas import tpu as pltpu


def _copy_body(x_ref, r_ref, d_ref, xo_ref, ro_ref, do_ref,
               sem_x, sem_r, sem_d):
    cx = pltpu.make_async_copy(x_ref, xo_ref, sem_x)
    cr = pltpu.make_async_copy(r_ref, ro_ref, sem_r)
    cd = pltpu.make_async_copy(d_ref, do_ref, sem_d)
    cx.start()
    cr.start()
    cd.start()
    cx.wait()
    cr.wait()
    cd.wait()


def kernel(sampled_point_xyz, sampled_point_ray_direction, sampled_point_distance):
    n = sampled_point_xyz.shape[0]
    x_flat = sampled_point_xyz.reshape(-1)
    r_flat = sampled_point_ray_direction.reshape(-1)
    pos, ray, dists = pl.pallas_call(
        _copy_body,
        in_specs=[
            pl.BlockSpec(memory_space=pl.ANY),
            pl.BlockSpec(memory_space=pl.ANY),
            pl.BlockSpec(memory_space=pl.ANY),
        ],
        out_specs=[
            pl.BlockSpec(memory_space=pl.ANY),
            pl.BlockSpec(memory_space=pl.ANY),
            pl.BlockSpec(memory_space=pl.ANY),
        ],
        out_shape=[
            jax.ShapeDtypeStruct((n * 3,), jnp.float32),
            jax.ShapeDtypeStruct((n * 3,), jnp.float32),
            jax.ShapeDtypeStruct((n,), jnp.float32),
        ],
        scratch_shapes=[
            pltpu.SemaphoreType.DMA,
            pltpu.SemaphoreType.DMA,
            pltpu.SemaphoreType.DMA,
        ],
    )(x_flat, r_flat, sampled_point_distance)
    return (pos.reshape(n, 3), ray.reshape(n, 3), dists)


# speedup vs baseline: 7.3796x; 1.0012x over previous
import jax
import jax.numpy as jnp
from jax.experimental import pallas as pl
from jax.experimental.pallas import tpu as pltpu


_K = 8


def _copy_body(x_ref, r_ref, d_ref, xo_ref, ro_ref, do_ref,
               sem_x, sem_r, sem_d):
    copies = []
    for src, dst, sems in ((x_ref, xo_ref, sem_x),
                           (r_ref, ro_ref, sem_r),
                           (d_ref, do_ref, sem_d)):
        chunk = src.shape[0] // _K
        for i in range(_K):
            sl = pl.ds(i * chunk, chunk)
            copies.append(pltpu.make_async_copy(src.at[sl], dst.at[sl], sems.at[i]))
    for c in copies:
        c.start()
    for c in copies:
        c.wait()


def kernel(sampled_point_xyz, sampled_point_ray_direction, sampled_point_distance):
    n = sampled_point_xyz.shape[0]
    x_flat = sampled_point_xyz.reshape(-1)
    r_flat = sampled_point_ray_direction.reshape(-1)
    pos, ray, dists = pl.pallas_call(
        _copy_body,
        in_specs=[
            pl.BlockSpec(memory_space=pl.ANY),
            pl.BlockSpec(memory_space=pl.ANY),
            pl.BlockSpec(memory_space=pl.ANY),
        ],
        out_specs=[
            pl.BlockSpec(memory_space=pl.ANY),
            pl.BlockSpec(memory_space=pl.ANY),
            pl.BlockSpec(memory_space=pl.ANY),
        ],
        out_shape=[
            jax.ShapeDtypeStruct((n * 3,), jnp.float32),
            jax.ShapeDtypeStruct((n * 3,), jnp.float32),
            jax.ShapeDtypeStruct((n,), jnp.float32),
        ],
        scratch_shapes=[
            pltpu.SemaphoreType.DMA((_K,)),
            pltpu.SemaphoreType.DMA((_K,)),
            pltpu.SemaphoreType.DMA((_K,)),
        ],
    )(x_flat, r_flat, sampled_point_distance)
    return (pos.reshape(n, 3), ray.reshape(n, 3), dists)


# speedup vs baseline: 20.2833x; 2.7486x over previous
import jax
import jax.numpy as jnp
from jax.experimental import pallas as pl
from jax.experimental.pallas import tpu as pltpu

_G = 512


def _copy_body(x_ref, r_ref, d_ref, xo_ref, ro_ref, do_ref):
    xo_ref[...] = x_ref[...]
    ro_ref[...] = r_ref[...]
    do_ref[...] = d_ref[...]


def kernel(sampled_point_xyz, sampled_point_ray_direction, sampled_point_distance):
    n = sampled_point_xyz.shape[0]
    b = n // _G
    pos, ray, dists = pl.pallas_call(
        _copy_body,
        grid=(_G,),
        in_specs=[
            pl.BlockSpec((b, 3), lambda i: (i, 0)),
            pl.BlockSpec((b, 3), lambda i: (i, 0)),
            pl.BlockSpec((b,), lambda i: (i,)),
        ],
        out_specs=[
            pl.BlockSpec((b, 3), lambda i: (i, 0)),
            pl.BlockSpec((b, 3), lambda i: (i, 0)),
            pl.BlockSpec((b,), lambda i: (i,)),
        ],
        out_shape=[
            jax.ShapeDtypeStruct((n, 3), jnp.float32),
            jax.ShapeDtypeStruct((n, 3), jnp.float32),
            jax.ShapeDtypeStruct((n,), jnp.float32),
        ],
    )(sampled_point_xyz, sampled_point_ray_direction, sampled_point_distance)
    return (pos, ray, dists)
